# baseline (speedup 1.0000x reference)
import functools

import jax
import jax.numpy as jnp
from jax import lax
from jax.experimental import pallas as pl
from jax.experimental.pallas import tpu as pltpu
from jax.experimental.pallas import tpu_sc as plsc

K = 16
R = 128
CH = 32


def _tc_body(x_ref, xt_ref, w1_ref, b1_ref, w2_ref, b2_ref, idx_ref, bias_ref):
    Rr = xt_ref.shape[1]
    N = x_ref.shape[2]
    H = bias_ref.shape[1]

    x = x_ref[0]
    xt = xt_ref[0]

    sq_row = x[0:1, :] * x[0:1, :] + x[1:2, :] * x[1:2, :] + x[2:3, :] * x[2:3, :]
    sq_col = jnp.sum(xt * xt, axis=1, keepdims=True)
    dot = jnp.dot(xt, x, preferred_element_type=jnp.float32)
    acc = sq_col + sq_row - 2.0 * dot

    jiota = lax.broadcasted_iota(jnp.int32, (Rr, N), 1)
    w1 = w1_ref[...]
    b1 = b1_ref[...]
    w2 = w2_ref[...]
    b2 = b2_ref[...]

    for k in range(K):
        m = jnp.min(acc, axis=1, keepdims=True)
        idxk = jnp.min(jnp.where(acc == m, jiota, N), axis=1, keepdims=True)
        maskb = jiota == idxk
        mask = maskb.astype(jnp.float32)
        acc = jnp.where(maskb, jnp.inf, acc)

        nx = jnp.sum(mask * x[0:1, :], axis=1, keepdims=True)
        ny = jnp.sum(mask * x[1:2, :], axis=1, keepdims=True)
        nz = jnp.sum(mask * x[2:3, :], axis=1, keepdims=True)
        relx = xt[:, 0:1] - nx
        rely = xt[:, 1:2] - ny
        relz = xt[:, 2:3] - nz

        hid = jnp.maximum(
            relx * w1[0:1, :] + rely * w1[1:2, :] + relz * w1[2:3, :] + b1, 0.0
        )
        biask = jnp.dot(hid, w2, preferred_element_type=jnp.float32) + b2

        idx_ref[0, :, k:k + 1] = idxk
        for h in range(bias_ref.shape[1]):
            bias_ref[0, h, :, k:k + 1] = biask[:, h:h + 1]


def _tc_stage(xyz, W1, b1, W2, b2):
    B, _, N = xyz.shape
    HID = W1.shape[1]
    H = W2.shape[1]
    xt = jnp.transpose(xyz, (0, 2, 1))
    b1r = b1.reshape(1, HID)
    b2r = b2.reshape(1, H)

    return pl.pallas_call(
        _tc_body,
        grid=(B, N // R),
        in_specs=[
            pl.BlockSpec((1, 3, N), lambda b, j: (b, 0, 0)),
            pl.BlockSpec((1, R, 3), lambda b, j: (b, j, 0)),
            pl.BlockSpec((3, HID), lambda b, j: (0, 0)),
            pl.BlockSpec((1, HID), lambda b, j: (0, 0)),
            pl.BlockSpec((HID, H), lambda b, j: (0, 0)),
            pl.BlockSpec((1, H), lambda b, j: (0, 0)),
        ],
        out_specs=[
            pl.BlockSpec((1, R, K), lambda b, j: (b, j, 0)),
            pl.BlockSpec((1, H, R, K), lambda b, j: (b, 0, j, 0)),
        ],
        out_shape=[
            jax.ShapeDtypeStruct((B, N, K), jnp.int32),
            jax.ShapeDtypeStruct((B, H, N, K), jnp.float32),
        ],
    )(xyz, xt, W1, b1r, W2, b2r)


def _sc_scatter(idx_f, bias_f, B, H, N):
    BH = B * H
    nchunks = N // CH
    mesh = plsc.VectorSubcoreMesh(core_axis_name="c", subcore_axis_name="s")
    zrows = jnp.zeros((CH * N,), jnp.float32)

    @functools.partial(
        pl.kernel,
        out_type=jax.ShapeDtypeStruct((BH * N * N,), jnp.float32),
        mesh=mesh,
        scratch_types=[
            pltpu.VMEM((CH * N,), jnp.float32),
            pltpu.VMEM((CH * K,), jnp.int32),
            pltpu.VMEM((CH * K,), jnp.float32),
        ],
        compiler_params=pltpu.CompilerParams(needs_layout_passes=False),
    )
    def k(idx_hbm, bias_hbm, z_hbm, out_hbm, buf, idxv, biasv):
        info = plsc.get_sparse_core_info()
        nc = info.num_cores
        wid = lax.axis_index("s") * nc + lax.axis_index("c")
        b = wid // H
        pltpu.sync_copy(z_hbm, buf)

        def chunk_body(c, _):
            i0 = c * CH
            pltpu.sync_copy(idx_hbm.at[pl.ds((b * N + i0) * K, CH * K)], idxv)
            pltpu.sync_copy(bias_hbm.at[pl.ds((wid * N + i0) * K, CH * K)], biasv)
            for r in range(CH):
                iv = idxv[pl.ds(r * K, K)] + (r * N)
                plsc.store_scatter(buf, [iv], biasv[pl.ds(r * K, K)])
            pltpu.sync_copy(buf, out_hbm.at[pl.ds((wid * N + i0) * N, CH * N)])
            zvec = jnp.zeros((K,), jnp.float32)
            for r in range(CH):
                iv = idxv[pl.ds(r * K, K)] + (r * N)
                plsc.store_scatter(buf, [iv], zvec)
            return 0

        lax.fori_loop(0, nchunks, chunk_body, 0)

    return k(idx_f, bias_f, zrows)


def kernel(xyz, W1, b1, W2, b2):
    B, _, N = xyz.shape
    H = W2.shape[1]
    idx, bias = _tc_stage(xyz, W1, b1, W2, b2)
    out = _sc_scatter(idx.reshape(-1), bias.reshape(-1), B, H, N)
    return out.reshape(B, H, N, N)
